# baseline (speedup 1.0000x reference)
import jax
import jax.numpy as jnp
from jax import lax
from jax.experimental import pallas as pl
from jax.experimental.pallas import tpu as pltpu
from jax.experimental.pallas import tpu_sc as plsc

N_NODES = 10000
N_EDGES = 160000
IN_DIM = 256
HID_DIM = 256
OUT_DIM = 128
HALF = 128

NC = 2
NT = 16
NPASS = 2
NQ = NC * NPASS
QCOL = HID_DIM // NQ
CHUNK = 224
ZC = 128
EPT = N_EDGES // NT
NCHUNK = (-(-EPT // CHUNK) + 3) // 4 * 4
PAD_E = NCHUNK * CHUNK - EPT
NROWS = 10112
NTRASH = NROWS - N_NODES
RPT = NROWS // NT
ZCHUNKS = RPT // ZC
ZTAIL = RPT - ZCHUNKS * ZC
NROWS_D = 10240
RPT_D = NROWS_D // NT
CHUNK_D = 128
NCHUNK_D = NCHUNK * CHUNK // CHUNK_D

R = 1000
NB = N_NODES // R

_MESH = plsc.VectorSubcoreMesh(core_axis_name="c", subcore_axis_name="s")


def _sc_deg_body(dstb, consts, deg_out, dst_v, zv, ov, deg_sh):
    c = lax.axis_index("c")
    s = lax.axis_index("s")
    base = s * RPT_D

    @pl.when(c == 0)
    def _stage():
        pltpu.sync_copy(dstb.at[s], dst_v)
        pltpu.sync_copy(consts.at[0, pl.ds(0, ZC)], zv)
        pltpu.sync_copy(consts.at[1, pl.ds(0, CHUNK_D)], ov)
        for j in range(RPT_D // ZC):
            pltpu.sync_copy(zv, deg_sh.at[pl.ds(base + j * ZC, ZC)])

    plsc.subcore_barrier()

    @pl.when(c == 0)
    def _accum():
        def body(j, carry):
            pltpu.sync_copy(ov, deg_sh.at[dst_v.at[j]], add=True)
            return carry
        lax.fori_loop(0, NCHUNK_D, body, 0)

    plsc.subcore_barrier()

    @pl.when(c == 0)
    def _out():
        pltpu.sync_copy(deg_sh.at[pl.ds(base, RPT_D)],
                        deg_out.at[pl.ds(base, RPT_D)])


_sc_deg = pl.kernel(
    _sc_deg_body,
    out_type=jax.ShapeDtypeStruct((NROWS_D,), jnp.float32),
    mesh=_MESH,
    scratch_types=[
        pltpu.VMEM((NCHUNK_D, CHUNK_D), jnp.int32),
        pltpu.VMEM((ZC,), jnp.float32),
        pltpu.VMEM((CHUNK_D,), jnp.float32),
        pltpu.VMEM_SHARED((NROWS_D,), jnp.float32),
    ],
)


def _sc_edge_body(ytab, srcb4, dstb, zrows, acc_out, src_v, dst_v, g0, g1,
                  g2, g3, zbuf, slab_sh, sem0, sem1, sem2, sem3, sem4, sem5,
                  sem6, sem7):
    c = lax.axis_index("c")
    s = lax.axis_index("s")
    base = s * RPT

    pltpu.sync_copy(dstb.at[s], dst_v)
    pltpu.sync_copy(zrows, zbuf)

    for p in range(NPASS):
        pltpu.sync_copy(srcb4.at[c, p, s], src_v)
        for j in range(ZCHUNKS):
            pltpu.sync_copy(zbuf, slab_sh.at[pl.ds(base + j * ZC, ZC)])
        pltpu.sync_copy(zbuf.at[pl.ds(0, ZTAIL)],
                        slab_sh.at[pl.ds(base + ZCHUNKS * ZC, ZTAIL)])

        plsc.subcore_barrier()

        gs = (g0, g1, g2, g3)
        gsem = (sem0, sem1, sem2, sem3)
        ssem = (sem4, sem5, sem6, sem7)

        def _gather(i, b):
            pltpu.async_copy(ytab.at[src_v.at[i]], gs[b], gsem[b])

        def _gather_wait(i, b):
            pltpu.make_async_copy(ytab.at[src_v.at[i]], gs[b], gsem[b]).wait()

        def _scatter(i, b):
            pltpu.async_copy(gs[b], slab_sh.at[dst_v.at[i]], ssem[b],
                             add=True)

        def _scatter_wait(i, b):
            pltpu.make_async_copy(gs[b], slab_sh.at[dst_v.at[i]],
                                  ssem[b]).wait()

        _gather(0, 0)
        _gather(1, 1)
        _gather_wait(0, 0); _scatter(0, 0); _gather(2, 2)
        _gather_wait(1, 1); _scatter(1, 1); _gather(3, 3)
        _gather_wait(2, 2); _scatter(2, 2); _scatter_wait(0, 0); _gather(4, 0)
        _gather_wait(3, 3); _scatter(3, 3); _scatter_wait(1, 1); _gather(5, 1)

        def body(gidx, carry):
            i = 4 * gidx
            for t in range(4):
                b2 = (t + 2) % 4
                _gather_wait(i + t, t)
                _scatter(i + t, t)
                _scatter_wait(i + t - 2, b2)
                _gather(i + t + 2, b2)
            return carry
        lax.fori_loop(1, NCHUNK // 4 - 1, body, 0)

        _gather_wait(NCHUNK - 4, 0); _scatter(NCHUNK - 4, 0)
        _scatter_wait(NCHUNK - 6, 2); _gather(NCHUNK - 2, 2)
        _gather_wait(NCHUNK - 3, 1); _scatter(NCHUNK - 3, 1)
        _scatter_wait(NCHUNK - 5, 3); _gather(NCHUNK - 1, 3)
        _gather_wait(NCHUNK - 2, 2); _scatter(NCHUNK - 2, 2)
        _scatter_wait(NCHUNK - 4, 0)
        _gather_wait(NCHUNK - 1, 3); _scatter(NCHUNK - 1, 3)
        _scatter_wait(NCHUNK - 3, 1)
        _scatter_wait(NCHUNK - 2, 2)
        _scatter_wait(NCHUNK - 1, 3)

        plsc.subcore_barrier()

        pltpu.sync_copy(
            slab_sh.at[pl.ds(base, RPT)],
            acc_out.at[p, pl.ds(base, RPT), pl.ds(c * QCOL, QCOL)])


_sc_edge = pl.kernel(
    _sc_edge_body,
    out_type=jax.ShapeDtypeStruct((NPASS, NROWS, HALF), jnp.float32),
    mesh=_MESH,
    scratch_types=[
        pltpu.VMEM((NCHUNK, CHUNK), jnp.int32),
        pltpu.VMEM((NCHUNK, CHUNK), jnp.int32),
        pltpu.VMEM((CHUNK, QCOL), jnp.float32),
        pltpu.VMEM((CHUNK, QCOL), jnp.float32),
        pltpu.VMEM((CHUNK, QCOL), jnp.float32),
        pltpu.VMEM((CHUNK, QCOL), jnp.float32),
        pltpu.VMEM((ZC, QCOL), jnp.float32),
        pltpu.VMEM_SHARED((NROWS, QCOL), jnp.float32),
        pltpu.SemaphoreType.DMA,
        pltpu.SemaphoreType.DMA,
        pltpu.SemaphoreType.DMA,
        pltpu.SemaphoreType.DMA,
        pltpu.SemaphoreType.DMA,
        pltpu.SemaphoreType.DMA,
        pltpu.SemaphoreType.DMA,
        pltpu.SemaphoreType.DMA,
    ],
    compiler_params=pltpu.CompilerParams(use_tc_tiling_on_sc=False),
)


def _dot(a, b):
    return jax.lax.dot_general(a, b, (((1,), (0,)), ((), ())),
                               precision=lax.Precision.DEFAULT,
                               preferred_element_type=jnp.float32)


def _tc1_body(x_ref, w1_ref, deg_ref, y_ref):
    dinv = jnp.transpose(1.0 / jnp.sqrt(deg_ref[0] + 1.0), (1, 0))
    o = _dot(x_ref[...], w1_ref[...])
    y_ref[0] = o[:, :HALF] * dinv
    y_ref[1] = o[:, HALF:] * dinv


_tc1 = pl.pallas_call(
    _tc1_body,
    grid=(NB,),
    in_specs=[
        pl.BlockSpec((R, IN_DIM), lambda i: (i, 0)),
        pl.BlockSpec((IN_DIM, HID_DIM), lambda i: (0, 0)),
        pl.BlockSpec((1, 1, R), lambda i: (i, 0, 0)),
    ],
    out_specs=pl.BlockSpec((NPASS, R, HALF), lambda i: (0, i, 0)),
    out_shape=jax.ShapeDtypeStruct((NPASS, N_NODES, HALF), jnp.float32),
)


def _tc2_body(acc_ref, y1_ref, deg_ref, b1_ref, w2_ref, y2_ref):
    dinv = jnp.transpose(1.0 / jnp.sqrt(deg_ref[0] + 1.0), (1, 0))
    h0 = jnp.maximum((acc_ref[0] + y1_ref[0]) * dinv + b1_ref[0], 0.0)
    h1 = jnp.maximum((acc_ref[1] + y1_ref[1]) * dinv + b1_ref[1], 0.0)
    o = _dot(h0, w2_ref[:HALF, :]) + _dot(h1, w2_ref[HALF:, :])
    y2_ref[0] = o[:, :HALF] * dinv
    y2_ref[1] = o[:, HALF:] * dinv


_tc2 = pl.pallas_call(
    _tc2_body,
    grid=(NB,),
    in_specs=[
        pl.BlockSpec((NPASS, R, HALF), lambda i: (0, i, 0)),
        pl.BlockSpec((NPASS, R, HALF), lambda i: (0, i, 0)),
        pl.BlockSpec((1, 1, R), lambda i: (i, 0, 0)),
        pl.BlockSpec((NPASS, 1, HALF), lambda i: (0, 0, 0)),
        pl.BlockSpec((HID_DIM, HID_DIM), lambda i: (0, 0)),
    ],
    out_specs=pl.BlockSpec((NPASS, R, HALF), lambda i: (0, i, 0)),
    out_shape=jax.ShapeDtypeStruct((NPASS, N_NODES, HALF), jnp.float32),
)


def _tc3_body(acc_ref, y2_ref, deg_ref, b2_ref, wfc_ref, bfc_ref, out_ref,
              g_ref):
    i = pl.program_id(0)
    dinv = jnp.transpose(1.0 / jnp.sqrt(deg_ref[0] + 1.0), (1, 0))
    h0 = jnp.maximum((acc_ref[0] + y2_ref[0]) * dinv + b2_ref[0], 0.0)
    h1 = jnp.maximum((acc_ref[1] + y2_ref[1]) * dinv + b2_ref[1], 0.0)
    bm = jnp.max(jnp.concatenate([h0, h1], axis=1), axis=0,
                 keepdims=True)

    @pl.when(i == 0)
    def _init():
        g_ref[...] = jnp.broadcast_to(bm, g_ref.shape)

    @pl.when(i > 0)
    def _acc():
        g_ref[...] = jnp.maximum(g_ref[...], bm)

    @pl.when(i == pl.num_programs(0) - 1)
    def _fin():
        g = jnp.max(g_ref[...], axis=0, keepdims=True)
        out_ref[...] = _dot(g, wfc_ref[...]) + bfc_ref[...]


_tc3 = pl.pallas_call(
    _tc3_body,
    grid=(NB,),
    in_specs=[
        pl.BlockSpec((NPASS, R, HALF), lambda i: (0, i, 0)),
        pl.BlockSpec((NPASS, R, HALF), lambda i: (0, i, 0)),
        pl.BlockSpec((1, 1, R), lambda i: (i, 0, 0)),
        pl.BlockSpec((NPASS, 1, HALF), lambda i: (0, 0, 0)),
        pl.BlockSpec((HID_DIM, OUT_DIM), lambda i: (0, 0)),
        pl.BlockSpec((1, OUT_DIM), lambda i: (0, 0)),
    ],
    out_specs=pl.BlockSpec((1, OUT_DIM), lambda i: (0, 0)),
    out_shape=jax.ShapeDtypeStruct((1, OUT_DIM), jnp.float32),
    scratch_shapes=[
        pltpu.VMEM((8, HID_DIM), jnp.float32),
    ],
)


def kernel(x, edge_index, W1, b1, W2, b2, Wfc, bfc):
    src = edge_index[0].astype(jnp.int32)
    dst = edge_index[1].astype(jnp.int32)

    pad_src = (jnp.arange(PAD_E, dtype=jnp.int32) * 89) % N_NODES
    pad_dst = N_NODES + jnp.arange(PAD_E, dtype=jnp.int32) % NTRASH
    srcp = jnp.concatenate(
        [src.reshape(NT, EPT), jnp.tile(pad_src[None], (NT, 1))],
        axis=1).reshape(NT, NCHUNK, CHUNK)
    dstb = jnp.concatenate(
        [dst.reshape(NT, EPT), jnp.tile(pad_dst[None], (NT, 1))],
        axis=1).reshape(NT, NCHUNK, CHUNK)
    srcb4 = jnp.stack(
        [jnp.stack([2 * (p * N_NODES + srcp) + c for p in range(NPASS)])
         for c in range(NC)])

    consts = jnp.stack([jnp.zeros((CHUNK,), jnp.float32),
                        jnp.ones((CHUNK,), jnp.float32)])
    zrows = jnp.zeros((ZC, QCOL), jnp.float32)

    deg = _sc_deg(dstb.reshape(NT, NCHUNK_D, CHUNK_D), consts)
    deg4 = deg[:N_NODES].reshape(NB, 1, R)

    y1 = _tc1(x, W1, deg4)
    acc1 = _sc_edge(y1.reshape(NPASS * N_NODES * 2, QCOL), srcb4, dstb, zrows)
    y2 = _tc2(acc1, y1, deg4, b1.reshape(NPASS, 1, HALF), W2)
    acc2 = _sc_edge(y2.reshape(NPASS * N_NODES * 2, QCOL), srcb4, dstb, zrows)
    out = _tc3(acc2, y2, deg4, b2.reshape(NPASS, 1, HALF), Wfc,
               bfc.reshape(1, OUT_DIM))
    return out.reshape(OUT_DIM)


# speedup vs baseline: 1.0789x; 1.0789x over previous
import jax
import jax.numpy as jnp
from jax import lax
from jax.experimental import pallas as pl
from jax.experimental.pallas import tpu as pltpu
from jax.experimental.pallas import tpu_sc as plsc

N_NODES = 10000
N_EDGES = 160000
IN_DIM = 256
HID_DIM = 256
OUT_DIM = 128
HALF = 128

NC = 2
NT = 16
NPASS = 2
NQ = NC * NPASS
QCOL = HID_DIM // NQ
CHUNK = 256
ZC = 128
EPT = N_EDGES // NT
NCHUNK = -(-EPT // CHUNK)
PAD_E = NCHUNK * CHUNK - EPT
NROWS = 10112
NTRASH = NROWS - N_NODES
RPT = NROWS // NT
ZCHUNKS = RPT // ZC
ZTAIL = RPT - ZCHUNKS * ZC
NROWS_D = 10240
RPT_D = NROWS_D // NT
CHUNK_D = 128
NCHUNK_D = NCHUNK * CHUNK // CHUNK_D

R = 1000
NB = N_NODES // R

_MESH = plsc.VectorSubcoreMesh(core_axis_name="c", subcore_axis_name="s")


def _sc_deg_body(dstb, consts, deg_out, dst_v, zv, ov, deg_sh):
    c = lax.axis_index("c")
    s = lax.axis_index("s")
    base = s * RPT_D

    pltpu.sync_copy(dstb.at[s, pl.ds(c * (NCHUNK_D // 2), NCHUNK_D // 2)],
                    dst_v)
    pltpu.sync_copy(consts.at[0, pl.ds(0, ZC)], zv)
    pltpu.sync_copy(consts.at[1, pl.ds(0, CHUNK_D)], ov)
    for j in range(RPT_D // ZC):
        pltpu.sync_copy(zv, deg_sh.at[pl.ds(base + j * ZC, ZC)])

    plsc.subcore_barrier()

    def body(j, carry):
        pltpu.sync_copy(ov, deg_sh.at[dst_v.at[j]], add=True)
        return carry
    lax.fori_loop(0, NCHUNK_D // 2, body, 0)

    plsc.subcore_barrier()

    pltpu.sync_copy(deg_sh.at[pl.ds(base, RPT_D)],
                    deg_out.at[c, pl.ds(base, RPT_D)])


_sc_deg = pl.kernel(
    _sc_deg_body,
    out_type=jax.ShapeDtypeStruct((NC, NROWS_D), jnp.float32),
    mesh=_MESH,
    scratch_types=[
        pltpu.VMEM((NCHUNK_D // 2, CHUNK_D), jnp.int32),
        pltpu.VMEM((ZC,), jnp.float32),
        pltpu.VMEM((CHUNK_D,), jnp.float32),
        pltpu.VMEM_SHARED((NROWS_D,), jnp.float32),
    ],
)


def _sc_edge_body(ytab, srcb4, dstb, zrows, acc_out, src_v, dst_v, g0, g1,
                  zbuf, slab_sh, sem0, sem1):
    c = lax.axis_index("c")
    s = lax.axis_index("s")
    base = s * RPT

    pltpu.sync_copy(dstb.at[s], dst_v)
    pltpu.sync_copy(zrows, zbuf)

    for p in range(NPASS):
        pltpu.sync_copy(srcb4.at[c, p, s], src_v)
        for j in range(ZCHUNKS):
            pltpu.sync_copy(zbuf, slab_sh.at[pl.ds(base + j * ZC, ZC)])
        pltpu.sync_copy(zbuf.at[pl.ds(0, ZTAIL)],
                        slab_sh.at[pl.ds(base + ZCHUNKS * ZC, ZTAIL)])

        plsc.subcore_barrier()

        pltpu.async_copy(ytab.at[src_v.at[0]], g0, sem0)

        def body(i, carry):
            j = 2 * i
            pltpu.async_copy(ytab.at[src_v.at[j + 1]], g1, sem1)
            pltpu.make_async_copy(ytab.at[src_v.at[j]], g0, sem0).wait()
            pltpu.sync_copy(g0, slab_sh.at[dst_v.at[j]], add=True)
            pltpu.async_copy(ytab.at[src_v.at[j + 2]], g0, sem0)
            pltpu.make_async_copy(ytab.at[src_v.at[j + 1]], g1, sem1).wait()
            pltpu.sync_copy(g1, slab_sh.at[dst_v.at[j + 1]], add=True)
            return carry
        lax.fori_loop(0, (NCHUNK - 1) // 2, body, 0)

        if NCHUNK % 2 == 0:
            pltpu.async_copy(ytab.at[src_v.at[NCHUNK - 1]], g1, sem1)
            pltpu.make_async_copy(ytab.at[src_v.at[NCHUNK - 2]], g0,
                                  sem0).wait()
            pltpu.sync_copy(g0, slab_sh.at[dst_v.at[NCHUNK - 2]], add=True)
            pltpu.make_async_copy(ytab.at[src_v.at[NCHUNK - 1]], g1,
                                  sem1).wait()
            pltpu.sync_copy(g1, slab_sh.at[dst_v.at[NCHUNK - 1]], add=True)
        else:
            pltpu.make_async_copy(ytab.at[src_v.at[NCHUNK - 1]], g0,
                                  sem0).wait()
            pltpu.sync_copy(g0, slab_sh.at[dst_v.at[NCHUNK - 1]], add=True)

        plsc.subcore_barrier()

        pltpu.sync_copy(
            slab_sh.at[pl.ds(base, RPT)],
            acc_out.at[p, pl.ds(base, RPT), pl.ds(c * QCOL, QCOL)])


_sc_edge = pl.kernel(
    _sc_edge_body,
    out_type=jax.ShapeDtypeStruct((NPASS, NROWS, HALF), jnp.float32),
    mesh=_MESH,
    scratch_types=[
        pltpu.VMEM((NCHUNK, CHUNK), jnp.int32),
        pltpu.VMEM((NCHUNK, CHUNK), jnp.int32),
        pltpu.VMEM((CHUNK, QCOL), jnp.float32),
        pltpu.VMEM((CHUNK, QCOL), jnp.float32),
        pltpu.VMEM((ZC, QCOL), jnp.float32),
        pltpu.VMEM_SHARED((NROWS, QCOL), jnp.float32),
        pltpu.SemaphoreType.DMA,
        pltpu.SemaphoreType.DMA,
    ],
    compiler_params=pltpu.CompilerParams(use_tc_tiling_on_sc=False),
)


def _dot(a, b):
    return jax.lax.dot_general(a, b, (((1,), (0,)), ((), ())),
                               precision=lax.Precision.DEFAULT,
                               preferred_element_type=jnp.float32)


def _tc1_body(x_ref, w1_ref, deg_ref, y_ref):
    dinv = jnp.transpose(1.0 / jnp.sqrt(deg_ref[0] + 1.0), (1, 0))
    o = _dot(x_ref[...], w1_ref[...])
    y_ref[0] = o[:, :HALF] * dinv
    y_ref[1] = o[:, HALF:] * dinv


_tc1 = pl.pallas_call(
    _tc1_body,
    grid=(NB,),
    in_specs=[
        pl.BlockSpec((R, IN_DIM), lambda i: (i, 0)),
        pl.BlockSpec((IN_DIM, HID_DIM), lambda i: (0, 0)),
        pl.BlockSpec((1, 1, R), lambda i: (i, 0, 0)),
    ],
    out_specs=pl.BlockSpec((NPASS, R, HALF), lambda i: (0, i, 0)),
    out_shape=jax.ShapeDtypeStruct((NPASS, N_NODES, HALF), jnp.float32),
)


def _tc2_body(acc_ref, y1_ref, deg_ref, b1_ref, w2_ref, y2_ref):
    dinv = jnp.transpose(1.0 / jnp.sqrt(deg_ref[0] + 1.0), (1, 0))
    h0 = jnp.maximum((acc_ref[0] + y1_ref[0]) * dinv + b1_ref[0], 0.0)
    h1 = jnp.maximum((acc_ref[1] + y1_ref[1]) * dinv + b1_ref[1], 0.0)
    o = _dot(h0, w2_ref[:HALF, :]) + _dot(h1, w2_ref[HALF:, :])
    y2_ref[0] = o[:, :HALF] * dinv
    y2_ref[1] = o[:, HALF:] * dinv


_tc2 = pl.pallas_call(
    _tc2_body,
    grid=(NB,),
    in_specs=[
        pl.BlockSpec((NPASS, R, HALF), lambda i: (0, i, 0)),
        pl.BlockSpec((NPASS, R, HALF), lambda i: (0, i, 0)),
        pl.BlockSpec((1, 1, R), lambda i: (i, 0, 0)),
        pl.BlockSpec((NPASS, 1, HALF), lambda i: (0, 0, 0)),
        pl.BlockSpec((HID_DIM, HID_DIM), lambda i: (0, 0)),
    ],
    out_specs=pl.BlockSpec((NPASS, R, HALF), lambda i: (0, i, 0)),
    out_shape=jax.ShapeDtypeStruct((NPASS, N_NODES, HALF), jnp.float32),
)


def _tc3_body(acc_ref, y2_ref, deg_ref, b2_ref, wfc_ref, bfc_ref, out_ref,
              g_ref):
    i = pl.program_id(0)
    dinv = jnp.transpose(1.0 / jnp.sqrt(deg_ref[0] + 1.0), (1, 0))
    h0 = jnp.maximum((acc_ref[0] + y2_ref[0]) * dinv + b2_ref[0], 0.0)
    h1 = jnp.maximum((acc_ref[1] + y2_ref[1]) * dinv + b2_ref[1], 0.0)
    bm = jnp.max(jnp.concatenate([h0, h1], axis=1), axis=0,
                 keepdims=True)

    @pl.when(i == 0)
    def _init():
        g_ref[...] = jnp.broadcast_to(bm, g_ref.shape)

    @pl.when(i > 0)
    def _acc():
        g_ref[...] = jnp.maximum(g_ref[...], bm)

    @pl.when(i == pl.num_programs(0) - 1)
    def _fin():
        g = jnp.max(g_ref[...], axis=0, keepdims=True)
        out_ref[...] = _dot(g, wfc_ref[...]) + bfc_ref[...]


_tc3 = pl.pallas_call(
    _tc3_body,
    grid=(NB,),
    in_specs=[
        pl.BlockSpec((NPASS, R, HALF), lambda i: (0, i, 0)),
        pl.BlockSpec((NPASS, R, HALF), lambda i: (0, i, 0)),
        pl.BlockSpec((1, 1, R), lambda i: (i, 0, 0)),
        pl.BlockSpec((NPASS, 1, HALF), lambda i: (0, 0, 0)),
        pl.BlockSpec((HID_DIM, OUT_DIM), lambda i: (0, 0)),
        pl.BlockSpec((1, OUT_DIM), lambda i: (0, 0)),
    ],
    out_specs=pl.BlockSpec((1, OUT_DIM), lambda i: (0, 0)),
    out_shape=jax.ShapeDtypeStruct((1, OUT_DIM), jnp.float32),
    scratch_shapes=[
        pltpu.VMEM((8, HID_DIM), jnp.float32),
    ],
)


def kernel(x, edge_index, W1, b1, W2, b2, Wfc, bfc):
    src = edge_index[0].astype(jnp.int32)
    dst = edge_index[1].astype(jnp.int32)

    pad_src = (jnp.arange(PAD_E, dtype=jnp.int32) * 89) % N_NODES
    pad_dst = N_NODES + jnp.arange(PAD_E, dtype=jnp.int32) % NTRASH
    srcp = jnp.concatenate(
        [src.reshape(NT, EPT), jnp.tile(pad_src[None], (NT, 1))],
        axis=1).reshape(NT, NCHUNK, CHUNK)
    dstb = jnp.concatenate(
        [dst.reshape(NT, EPT), jnp.tile(pad_dst[None], (NT, 1))],
        axis=1).reshape(NT, NCHUNK, CHUNK)
    srcb4 = jnp.stack(
        [jnp.stack([2 * (p * N_NODES + srcp) + c for p in range(NPASS)])
         for c in range(NC)])

    consts = jnp.stack([jnp.zeros((CHUNK,), jnp.float32),
                        jnp.ones((CHUNK,), jnp.float32)])
    zrows = jnp.zeros((ZC, QCOL), jnp.float32)

    degp = _sc_deg(dstb.reshape(NT, NCHUNK_D, CHUNK_D), consts)
    deg = degp[0] + degp[1]
    deg4 = deg[:N_NODES].reshape(NB, 1, R)

    y1 = _tc1(x, W1, deg4)
    acc1 = _sc_edge(y1.reshape(NPASS * N_NODES * 2, QCOL), srcb4, dstb, zrows)
    y2 = _tc2(acc1, y1, deg4, b1.reshape(NPASS, 1, HALF), W2)
    acc2 = _sc_edge(y2.reshape(NPASS * N_NODES * 2, QCOL), srcb4, dstb, zrows)
    out = _tc3(acc2, y2, deg4, b2.reshape(NPASS, 1, HALF), Wfc,
               bfc.reshape(1, OUT_DIM))
    return out.reshape(OUT_DIM)


# speedup vs baseline: 1.0852x; 1.0058x over previous
import jax
import jax.numpy as jnp
from jax import lax
from jax.experimental import pallas as pl
from jax.experimental.pallas import tpu as pltpu
from jax.experimental.pallas import tpu_sc as plsc

N_NODES = 10000
N_EDGES = 160000
IN_DIM = 256
HID_DIM = 256
OUT_DIM = 128
HALF = 128

NC = 2
NT = 16
NPASS = 2
NQ = NC * NPASS
QCOL = HID_DIM // NQ
CHUNK = 320
ZC = 128
EPT = N_EDGES // NT
NCHUNK = -(-EPT // CHUNK)
PAD_E = NCHUNK * CHUNK - EPT
NROWS = 10112
NTRASH = NROWS - N_NODES
RPT = NROWS // NT
ZCHUNKS = RPT // ZC
ZTAIL = RPT - ZCHUNKS * ZC
NROWS_D = 10240
RPT_D = NROWS_D // NT
CHUNK_D = 128
NCHUNK_D = NCHUNK * CHUNK // CHUNK_D

R = 1000
NB = N_NODES // R

_MESH = plsc.VectorSubcoreMesh(core_axis_name="c", subcore_axis_name="s")


def _sc_deg_body(dstb, consts, deg_out, dst_v, zv, ov, deg_sh):
    c = lax.axis_index("c")
    s = lax.axis_index("s")
    base = s * RPT_D

    pltpu.sync_copy(dstb.at[s, pl.ds(c * (NCHUNK_D // 2), NCHUNK_D // 2)],
                    dst_v)
    pltpu.sync_copy(consts.at[0, pl.ds(0, ZC)], zv)
    pltpu.sync_copy(consts.at[1, pl.ds(0, CHUNK_D)], ov)
    for j in range(RPT_D // ZC):
        pltpu.sync_copy(zv, deg_sh.at[pl.ds(base + j * ZC, ZC)])

    plsc.subcore_barrier()

    def body(j, carry):
        pltpu.sync_copy(ov, deg_sh.at[dst_v.at[j]], add=True)
        return carry
    lax.fori_loop(0, NCHUNK_D // 2, body, 0)

    plsc.subcore_barrier()

    pltpu.sync_copy(deg_sh.at[pl.ds(base, RPT_D)],
                    deg_out.at[c, pl.ds(base, RPT_D)])


_sc_deg = pl.kernel(
    _sc_deg_body,
    out_type=jax.ShapeDtypeStruct((NC, NROWS_D), jnp.float32),
    mesh=_MESH,
    scratch_types=[
        pltpu.VMEM((NCHUNK_D // 2, CHUNK_D), jnp.int32),
        pltpu.VMEM((ZC,), jnp.float32),
        pltpu.VMEM((CHUNK_D,), jnp.float32),
        pltpu.VMEM_SHARED((NROWS_D,), jnp.float32),
    ],
)


def _sc_edge_body(ytab, srcb4, dstb, zrows, acc_out, src_v, dst_v, g0, g1,
                  zbuf, slab_sh, sem0, sem1):
    c = lax.axis_index("c")
    s = lax.axis_index("s")
    base = s * RPT

    pltpu.sync_copy(dstb.at[s], dst_v)
    pltpu.sync_copy(zrows, zbuf)

    for p in range(NPASS):
        pltpu.sync_copy(srcb4.at[c, p, s], src_v)
        for j in range(ZCHUNKS):
            pltpu.sync_copy(zbuf, slab_sh.at[pl.ds(base + j * ZC, ZC)])
        pltpu.sync_copy(zbuf.at[pl.ds(0, ZTAIL)],
                        slab_sh.at[pl.ds(base + ZCHUNKS * ZC, ZTAIL)])

        plsc.subcore_barrier()

        pltpu.async_copy(ytab.at[src_v.at[0]], g0, sem0)

        def body(i, carry):
            j = 2 * i
            pltpu.async_copy(ytab.at[src_v.at[j + 1]], g1, sem1)
            pltpu.make_async_copy(ytab.at[src_v.at[j]], g0, sem0).wait()
            pltpu.sync_copy(g0, slab_sh.at[dst_v.at[j]], add=True)
            pltpu.async_copy(ytab.at[src_v.at[j + 2]], g0, sem0)
            pltpu.make_async_copy(ytab.at[src_v.at[j + 1]], g1, sem1).wait()
            pltpu.sync_copy(g1, slab_sh.at[dst_v.at[j + 1]], add=True)
            return carry
        lax.fori_loop(0, (NCHUNK - 1) // 2, body, 0)

        if NCHUNK % 2 == 0:
            pltpu.async_copy(ytab.at[src_v.at[NCHUNK - 1]], g1, sem1)
            pltpu.make_async_copy(ytab.at[src_v.at[NCHUNK - 2]], g0,
                                  sem0).wait()
            pltpu.sync_copy(g0, slab_sh.at[dst_v.at[NCHUNK - 2]], add=True)
            pltpu.make_async_copy(ytab.at[src_v.at[NCHUNK - 1]], g1,
                                  sem1).wait()
            pltpu.sync_copy(g1, slab_sh.at[dst_v.at[NCHUNK - 1]], add=True)
        else:
            pltpu.make_async_copy(ytab.at[src_v.at[NCHUNK - 1]], g0,
                                  sem0).wait()
            pltpu.sync_copy(g0, slab_sh.at[dst_v.at[NCHUNK - 1]], add=True)

        plsc.subcore_barrier()

        pltpu.sync_copy(
            slab_sh.at[pl.ds(base, RPT)],
            acc_out.at[p, pl.ds(base, RPT), pl.ds(c * QCOL, QCOL)])


_sc_edge = pl.kernel(
    _sc_edge_body,
    out_type=jax.ShapeDtypeStruct((NPASS, NROWS, HALF), jnp.float32),
    mesh=_MESH,
    scratch_types=[
        pltpu.VMEM((NCHUNK, CHUNK), jnp.int32),
        pltpu.VMEM((NCHUNK, CHUNK), jnp.int32),
        pltpu.VMEM((CHUNK, QCOL), jnp.float32),
        pltpu.VMEM((CHUNK, QCOL), jnp.float32),
        pltpu.VMEM((ZC, QCOL), jnp.float32),
        pltpu.VMEM_SHARED((NROWS, QCOL), jnp.float32),
        pltpu.SemaphoreType.DMA,
        pltpu.SemaphoreType.DMA,
    ],
    compiler_params=pltpu.CompilerParams(use_tc_tiling_on_sc=False),
)


def _dot(a, b):
    return jax.lax.dot_general(a, b, (((1,), (0,)), ((), ())),
                               precision=lax.Precision.DEFAULT,
                               preferred_element_type=jnp.float32)


def _tc1_body(x_ref, w1_ref, deg_ref, y_ref):
    dinv = jnp.transpose(1.0 / jnp.sqrt(deg_ref[0] + 1.0), (1, 0))
    o = _dot(x_ref[...], w1_ref[...])
    y_ref[0] = o[:, :HALF] * dinv
    y_ref[1] = o[:, HALF:] * dinv


_tc1 = pl.pallas_call(
    _tc1_body,
    grid=(NB,),
    in_specs=[
        pl.BlockSpec((R, IN_DIM), lambda i: (i, 0)),
        pl.BlockSpec((IN_DIM, HID_DIM), lambda i: (0, 0)),
        pl.BlockSpec((1, 1, R), lambda i: (i, 0, 0)),
    ],
    out_specs=pl.BlockSpec((NPASS, R, HALF), lambda i: (0, i, 0)),
    out_shape=jax.ShapeDtypeStruct((NPASS, N_NODES, HALF), jnp.float32),
)


def _tc2_body(acc_ref, y1_ref, deg_ref, b1_ref, w2_ref, y2_ref):
    dinv = jnp.transpose(1.0 / jnp.sqrt(deg_ref[0] + 1.0), (1, 0))
    h0 = jnp.maximum((acc_ref[0] + y1_ref[0]) * dinv + b1_ref[0], 0.0)
    h1 = jnp.maximum((acc_ref[1] + y1_ref[1]) * dinv + b1_ref[1], 0.0)
    o = _dot(h0, w2_ref[:HALF, :]) + _dot(h1, w2_ref[HALF:, :])
    y2_ref[0] = o[:, :HALF] * dinv
    y2_ref[1] = o[:, HALF:] * dinv


_tc2 = pl.pallas_call(
    _tc2_body,
    grid=(NB,),
    in_specs=[
        pl.BlockSpec((NPASS, R, HALF), lambda i: (0, i, 0)),
        pl.BlockSpec((NPASS, R, HALF), lambda i: (0, i, 0)),
        pl.BlockSpec((1, 1, R), lambda i: (i, 0, 0)),
        pl.BlockSpec((NPASS, 1, HALF), lambda i: (0, 0, 0)),
        pl.BlockSpec((HID_DIM, HID_DIM), lambda i: (0, 0)),
    ],
    out_specs=pl.BlockSpec((NPASS, R, HALF), lambda i: (0, i, 0)),
    out_shape=jax.ShapeDtypeStruct((NPASS, N_NODES, HALF), jnp.float32),
)


def _tc3_body(acc_ref, y2_ref, deg_ref, b2_ref, wfc_ref, bfc_ref, out_ref,
              g_ref):
    i = pl.program_id(0)
    dinv = jnp.transpose(1.0 / jnp.sqrt(deg_ref[0] + 1.0), (1, 0))
    h0 = jnp.maximum((acc_ref[0] + y2_ref[0]) * dinv + b2_ref[0], 0.0)
    h1 = jnp.maximum((acc_ref[1] + y2_ref[1]) * dinv + b2_ref[1], 0.0)
    bm = jnp.max(jnp.concatenate([h0, h1], axis=1), axis=0,
                 keepdims=True)

    @pl.when(i == 0)
    def _init():
        g_ref[...] = jnp.broadcast_to(bm, g_ref.shape)

    @pl.when(i > 0)
    def _acc():
        g_ref[...] = jnp.maximum(g_ref[...], bm)

    @pl.when(i == pl.num_programs(0) - 1)
    def _fin():
        g = jnp.max(g_ref[...], axis=0, keepdims=True)
        out_ref[...] = _dot(g, wfc_ref[...]) + bfc_ref[...]


_tc3 = pl.pallas_call(
    _tc3_body,
    grid=(NB,),
    in_specs=[
        pl.BlockSpec((NPASS, R, HALF), lambda i: (0, i, 0)),
        pl.BlockSpec((NPASS, R, HALF), lambda i: (0, i, 0)),
        pl.BlockSpec((1, 1, R), lambda i: (i, 0, 0)),
        pl.BlockSpec((NPASS, 1, HALF), lambda i: (0, 0, 0)),
        pl.BlockSpec((HID_DIM, OUT_DIM), lambda i: (0, 0)),
        pl.BlockSpec((1, OUT_DIM), lambda i: (0, 0)),
    ],
    out_specs=pl.BlockSpec((1, OUT_DIM), lambda i: (0, 0)),
    out_shape=jax.ShapeDtypeStruct((1, OUT_DIM), jnp.float32),
    scratch_shapes=[
        pltpu.VMEM((8, HID_DIM), jnp.float32),
    ],
)


def kernel(x, edge_index, W1, b1, W2, b2, Wfc, bfc):
    src = edge_index[0].astype(jnp.int32)
    dst = edge_index[1].astype(jnp.int32)

    pad_src = (jnp.arange(PAD_E, dtype=jnp.int32) * 89) % N_NODES
    pad_dst = N_NODES + jnp.arange(PAD_E, dtype=jnp.int32) % NTRASH
    srcp = jnp.concatenate(
        [src.reshape(NT, EPT), jnp.tile(pad_src[None], (NT, 1))],
        axis=1).reshape(NT, NCHUNK, CHUNK)
    dstb = jnp.concatenate(
        [dst.reshape(NT, EPT), jnp.tile(pad_dst[None], (NT, 1))],
        axis=1).reshape(NT, NCHUNK, CHUNK)
    srcb4 = jnp.stack(
        [jnp.stack([2 * (p * N_NODES + srcp) + c for p in range(NPASS)])
         for c in range(NC)])

    consts = jnp.stack([jnp.zeros((CHUNK,), jnp.float32),
                        jnp.ones((CHUNK,), jnp.float32)])
    zrows = jnp.zeros((ZC, QCOL), jnp.float32)

    degp = _sc_deg(dstb.reshape(NT, NCHUNK_D, CHUNK_D), consts)
    deg = degp[0] + degp[1]
    deg4 = deg[:N_NODES].reshape(NB, 1, R)

    y1 = _tc1(x, W1, deg4)
    acc1 = _sc_edge(y1.reshape(NPASS * N_NODES * 2, QCOL), srcb4, dstb, zrows)
    y2 = _tc2(acc1, y1, deg4, b1.reshape(NPASS, 1, HALF), W2)
    acc2 = _sc_edge(y2.reshape(NPASS * N_NODES * 2, QCOL), srcb4, dstb, zrows)
    out = _tc3(acc2, y2, deg4, b2.reshape(NPASS, 1, HALF), Wfc,
               bfc.reshape(1, OUT_DIM))
    return out.reshape(OUT_DIM)
